# Initial kernel scaffold; baseline (speedup 1.0000x reference)
import jax
import jax.numpy as jnp
from jax.experimental import pallas as pl


def kernel(node_neighbors, edge_neighbors, node_feats, edge_feats, We, be, Wn, bn, Wce, bce, Wcn, bcn, ln_g, ln_b):
    raise NotImplementedError("write your pallas kernel here")


# speedup vs baseline: 2.9064x; 2.9064x over previous
import functools

import jax
import jax.numpy as jnp
from jax import lax
from jax.experimental import pallas as pl
from jax.experimental.pallas import tpu as pltpu
from jax.experimental.pallas import tpu_sc as plsc

N = 10000
DEG = 16
E = 160000
EDEG = 4
DN = 128
DE = 16
H = 128

NTILES = 32

ECHUNK = 128
E_PAD = 163840
E_CHUNKS_PER_TILE = E_PAD // ECHUNK // NTILES

NCHUNK = 128
N_PAD = 10240
N_CHUNKS = N_PAD // NCHUNK

_mesh = plsc.VectorSubcoreMesh(core_axis_name="c", subcore_axis_name="s")


def _wid():
    return lax.axis_index("c") * 16 + lax.axis_index("s")


def _edge_gather_body(ef_hbm, ent_hbm, out_hbm, idx_v, own_v, bufs, acc_v, sem):
    wid = _wid()

    def chunk_body(i, _):
        base = (wid * E_CHUNKS_PER_TILE + i) * ECHUNK
        pltpu.sync_copy(ent_hbm.at[:, pl.ds(base, ECHUNK)], idx_v)
        descs = [pltpu.async_copy(ef_hbm.at[pl.ds(base, ECHUNK)], own_v, sem)]
        for j in range(EDEG):
            descs.append(pltpu.async_copy(ef_hbm.at[idx_v.at[j]], bufs[j], sem))
        for d in descs:
            d.wait()

        def row_body(r, _):
            for u in range(8):
                rr = r * 8 + u
                acc_v[rr, :] = (own_v[rr, :] + bufs[0][rr, :] + bufs[1][rr, :]
                                + bufs[2][rr, :] + bufs[3][rr, :])
            return _

        lax.fori_loop(0, ECHUNK // 8, row_body, None)
        pltpu.sync_copy(acc_v, out_hbm.at[pl.ds(base, ECHUNK)])
        return _

    lax.fori_loop(0, E_CHUNKS_PER_TILE, chunk_body, None)


@functools.partial(
    pl.kernel,
    out_type=jax.ShapeDtypeStruct((E_PAD, DE), jnp.float32),
    mesh=_mesh,
    scratch_types=[
        pltpu.VMEM((EDEG, ECHUNK), jnp.int32),
        pltpu.VMEM((ECHUNK, DE), jnp.float32),
        [pltpu.VMEM((ECHUNK, DE), jnp.float32) for _ in range(EDEG)],
        pltpu.VMEM((ECHUNK, DE), jnp.float32),
        pltpu.SemaphoreType.DMA,
    ],
    compiler_params=pltpu.CompilerParams(use_tc_tiling_on_sc=False),
)
def _edge_gather(ef_hbm, ent_hbm, out_hbm, idx_v, own_v, bufs, acc_v, sem):
    _edge_gather_body(ef_hbm, ent_hbm, out_hbm, idx_v, own_v, bufs, acc_v, sem)


def _node_gather_body(nf_hbm, ne_tab_hbm, nnt_hbm, net_hbm, outn_hbm, oute_hbm,
                      idxn_v, idxe_v, own_v, bufn_v, bufe_v, accn_v, acce_v, sem):
    wid = _wid()

    def chunk_body(i, _):
        t = i * NTILES + wid

        @pl.when(t < N_CHUNKS)
        def _do():
            base = t * NCHUNK
            pltpu.sync_copy(nnt_hbm.at[:, pl.ds(base, NCHUNK)], idxn_v)
            pltpu.sync_copy(net_hbm.at[:, pl.ds(base, NCHUNK)], idxe_v)
            pltpu.async_copy(nf_hbm.at[pl.ds(base, NCHUNK)], own_v, sem).wait()

            for j in range(DEG):
                dn = pltpu.async_copy(nf_hbm.at[idxn_v.at[j]], bufn_v, sem)
                de = pltpu.async_copy(ne_tab_hbm.at[idxe_v.at[j]], bufe_v, sem)
                dn.wait()
                de.wait()

                def row_body(r, _):
                    for u in range(8):
                        s = pl.ds(u * 16, 16)
                        if j == 0:
                            accn_v[r, s] = own_v[r, s] + bufn_v[r, s]
                            acce_v[r, s] = bufe_v[r, s]
                        else:
                            accn_v[r, s] = accn_v[r, s] + bufn_v[r, s]
                            acce_v[r, s] = acce_v[r, s] + bufe_v[r, s]
                    return _

                lax.fori_loop(0, NCHUNK, row_body, None)

            pltpu.sync_copy(accn_v, outn_hbm.at[pl.ds(base, NCHUNK)])
            pltpu.sync_copy(acce_v, oute_hbm.at[pl.ds(base, NCHUNK)])

        return _

    lax.fori_loop(0, (N_CHUNKS + NTILES - 1) // NTILES, chunk_body, None)


@functools.partial(
    pl.kernel,
    out_type=(jax.ShapeDtypeStruct((N_PAD, DN), jnp.float32),
              jax.ShapeDtypeStruct((N_PAD, H), jnp.float32)),
    mesh=_mesh,
    scratch_types=[
        pltpu.VMEM((DEG, NCHUNK), jnp.int32),
        pltpu.VMEM((DEG, NCHUNK), jnp.int32),
        pltpu.VMEM((NCHUNK, DN), jnp.float32),
        pltpu.VMEM((NCHUNK, DN), jnp.float32),
        pltpu.VMEM((NCHUNK, H), jnp.float32),
        pltpu.VMEM((NCHUNK, DN), jnp.float32),
        pltpu.VMEM((NCHUNK, H), jnp.float32),
        pltpu.SemaphoreType.DMA,
    ],
)
def _node_gather(nf_hbm, ne_tab_hbm, nnt_hbm, net_hbm, outn_hbm, oute_hbm,
                 idxn_v, idxe_v, own_v, bufn_v, bufe_v, accn_v, acce_v, sem):
    _node_gather_body(nf_hbm, ne_tab_hbm, nnt_hbm, net_hbm, outn_hbm, oute_hbm,
                      idxn_v, idxe_v, own_v, bufn_v, bufe_v, accn_v, acce_v, sem)


def _mlp_ln_kernel(nsum, x_ref, w1_ref, b1_ref, w2_ref, b2_ref, g_ref, b_ref,
                   o_ref):
    h1 = jnp.dot(x_ref[...], w1_ref[...],
                 preferred_element_type=jnp.float32) + nsum * b1_ref[...]
    h2 = jax.nn.relu(jnp.dot(h1, w2_ref[...],
                             preferred_element_type=jnp.float32) + b2_ref[...])
    m = jnp.mean(h2, axis=-1, keepdims=True)
    v = jnp.mean((h2 - m) ** 2, axis=-1, keepdims=True)
    o_ref[...] = (h2 - m) / jnp.sqrt(v + 1e-5) * g_ref[...] + b_ref[...]


def _edge_mlp(agg_ef, WeT, be2, WceT, bce2, g2, b2):
    blk = 1000
    grid = E // blk
    return pl.pallas_call(
        functools.partial(_mlp_ln_kernel, float(EDEG + 1)),
        grid=(grid,),
        in_specs=[
            pl.BlockSpec((blk, DE), lambda i: (i, 0)),
            pl.BlockSpec((DE, H), lambda i: (0, 0)),
            pl.BlockSpec((1, H), lambda i: (0, 0)),
            pl.BlockSpec((H, H), lambda i: (0, 0)),
            pl.BlockSpec((1, H), lambda i: (0, 0)),
            pl.BlockSpec((1, H), lambda i: (0, 0)),
            pl.BlockSpec((1, H), lambda i: (0, 0)),
        ],
        out_specs=pl.BlockSpec((blk, H), lambda i: (i, 0)),
        out_shape=jax.ShapeDtypeStruct((E, H), jnp.float32),
    )(agg_ef, WeT, be2, WceT, bce2, g2, b2)


def _node_mlp_kernel(x_ref, e_ref, w1_ref, b1_ref, w2_ref, b2_ref, g_ref,
                     b_ref, o_ref):
    h1 = (jnp.dot(x_ref[...], w1_ref[...], preferred_element_type=jnp.float32)
          + float(DEG + 1) * b1_ref[...] + e_ref[...])
    h2 = jax.nn.relu(jnp.dot(h1, w2_ref[...],
                             preferred_element_type=jnp.float32) + b2_ref[...])
    m = jnp.mean(h2, axis=-1, keepdims=True)
    v = jnp.mean((h2 - m) ** 2, axis=-1, keepdims=True)
    o_ref[...] = (h2 - m) / jnp.sqrt(v + 1e-5) * g_ref[...] + b_ref[...]


def _node_mlp(agg_nf, agg_ne, WnT, bn2, WcnT, bcn2, g2, b2):
    blk = 1000
    grid = N // blk
    return pl.pallas_call(
        _node_mlp_kernel,
        grid=(grid,),
        in_specs=[
            pl.BlockSpec((blk, DN), lambda i: (i, 0)),
            pl.BlockSpec((blk, H), lambda i: (i, 0)),
            pl.BlockSpec((DN, H), lambda i: (0, 0)),
            pl.BlockSpec((1, H), lambda i: (0, 0)),
            pl.BlockSpec((H, H), lambda i: (0, 0)),
            pl.BlockSpec((1, H), lambda i: (0, 0)),
            pl.BlockSpec((1, H), lambda i: (0, 0)),
            pl.BlockSpec((1, H), lambda i: (0, 0)),
        ],
        out_specs=pl.BlockSpec((blk, H), lambda i: (i, 0)),
        out_shape=jax.ShapeDtypeStruct((N, H), jnp.float32),
    )(agg_nf, agg_ne, WnT, bn2, WcnT, bcn2, g2, b2)


def kernel(node_neighbors, edge_neighbors, node_feats, edge_feats,
           We, be, Wn, bn, Wce, bce, Wcn, bcn, ln_g, ln_b):
    en_t = edge_neighbors.astype(jnp.int32).T
    en_t = jnp.pad(en_t, ((0, 0), (0, E_PAD - E)))
    nn_t = node_neighbors[:, :, 0].astype(jnp.int32).T
    ne_t = node_neighbors[:, :, 1].astype(jnp.int32).T
    nn_t = jnp.pad(nn_t, ((0, 0), (0, N_PAD - N)))
    ne_t = jnp.pad(ne_t, ((0, 0), (0, N_PAD - N)))
    ef_pad = jnp.pad(edge_feats, ((0, E_PAD - E), (0, 0)))
    nf_pad = jnp.pad(node_feats, ((0, N_PAD - N), (0, 0)))

    be2 = be.reshape(1, H)
    bn2 = bn.reshape(1, H)
    bce2 = bce.reshape(1, H)
    bcn2 = bcn.reshape(1, H)
    g2 = ln_g.reshape(1, H)
    b2 = ln_b.reshape(1, H)

    agg_ef = _edge_gather(ef_pad, en_t)[:E]

    new_edge = _edge_mlp(agg_ef, We.T, be2, Wce.T, bce2, g2, b2)

    agg_nf, agg_ne = _node_gather(nf_pad, new_edge, nn_t, ne_t)

    new_node = _node_mlp(agg_nf[:N], agg_ne[:N], Wn.T, bn2, Wcn.T, bcn2,
                         g2, b2)

    return (new_node, new_edge)


# speedup vs baseline: 3.2010x; 1.1014x over previous
import functools

import jax
import jax.numpy as jnp
from jax import lax
from jax.experimental import pallas as pl
from jax.experimental.pallas import tpu as pltpu
from jax.experimental.pallas import tpu_sc as plsc

N = 10000
DEG = 16
E = 160000
EDEG = 4
DN = 128
DE = 16
H = 128

NTILES = 32

ECHUNK = 128
E_PAD = 163840
E_CHUNKS_PER_TILE = E_PAD // ECHUNK // NTILES

NCHUNK = 128
N_PAD = 10240
N_CHUNKS = N_PAD // NCHUNK

_mesh = plsc.VectorSubcoreMesh(core_axis_name="c", subcore_axis_name="s")


def _wid():
    return lax.axis_index("c") * 16 + lax.axis_index("s")


def _edge_gather_body(ef_hbm, ent_hbm, out_hbm, idx_v, acc_v, sem):
    wid = _wid()

    def chunk_body(i, _):
        base = (wid * E_CHUNKS_PER_TILE + i) * ECHUNK
        pltpu.sync_copy(ent_hbm.at[:, pl.ds(base, ECHUNK)], idx_v)
        pltpu.sync_copy(ef_hbm.at[pl.ds(base, ECHUNK)], acc_v)
        descs = [pltpu.async_copy(ef_hbm.at[idx_v.at[j]], acc_v, sem, add=True)
                 for j in range(EDEG)]
        for d in descs:
            d.wait()
        pltpu.sync_copy(acc_v, out_hbm.at[pl.ds(base, ECHUNK)])
        return _

    lax.fori_loop(0, E_CHUNKS_PER_TILE, chunk_body, None)


@functools.partial(
    pl.kernel,
    out_type=jax.ShapeDtypeStruct((E_PAD, DE), jnp.float32),
    mesh=_mesh,
    scratch_types=[
        pltpu.VMEM((EDEG, ECHUNK), jnp.int32),
        pltpu.VMEM((ECHUNK, DE), jnp.float32),
        pltpu.SemaphoreType.DMA,
    ],
    compiler_params=pltpu.CompilerParams(use_tc_tiling_on_sc=False),
)
def _edge_gather(ef_hbm, ent_hbm, out_hbm, idx_v, acc_v, sem):
    _edge_gather_body(ef_hbm, ent_hbm, out_hbm, idx_v, acc_v, sem)


def _node_gather_body(nf_hbm, ne_tab_hbm, nnt_hbm, net_hbm, outn_hbm, oute_hbm,
                      idxn_v, idxe_v, accn_v, acce_v, sem):
    wid = _wid()

    def chunk_body(i, _):
        t = i * NTILES + wid

        @pl.when(t < N_CHUNKS)
        def _do():
            base = t * NCHUNK
            pltpu.sync_copy(nnt_hbm.at[:, pl.ds(base, NCHUNK)], idxn_v)
            pltpu.sync_copy(net_hbm.at[:, pl.ds(base, NCHUNK)], idxe_v)
            d0 = pltpu.async_copy(nf_hbm.at[pl.ds(base, NCHUNK)], accn_v, sem)
            d1 = pltpu.async_copy(ne_tab_hbm.at[idxe_v.at[0]], acce_v, sem)
            d0.wait()
            d1.wait()
            descs = []
            for j in range(DEG):
                descs.append(pltpu.async_copy(
                    nf_hbm.at[idxn_v.at[j]], accn_v, sem, add=True))
                if j > 0:
                    descs.append(pltpu.async_copy(
                        ne_tab_hbm.at[idxe_v.at[j]], acce_v, sem, add=True))
            for d in descs:
                d.wait()
            pltpu.sync_copy(accn_v, outn_hbm.at[pl.ds(base, NCHUNK)])
            pltpu.sync_copy(acce_v, oute_hbm.at[pl.ds(base, NCHUNK)])

        return _

    lax.fori_loop(0, (N_CHUNKS + NTILES - 1) // NTILES, chunk_body, None)


@functools.partial(
    pl.kernel,
    out_type=(jax.ShapeDtypeStruct((N_PAD, DN), jnp.float32),
              jax.ShapeDtypeStruct((N_PAD, H), jnp.float32)),
    mesh=_mesh,
    scratch_types=[
        pltpu.VMEM((DEG, NCHUNK), jnp.int32),
        pltpu.VMEM((DEG, NCHUNK), jnp.int32),
        pltpu.VMEM((NCHUNK, DN), jnp.float32),
        pltpu.VMEM((NCHUNK, H), jnp.float32),
        pltpu.SemaphoreType.DMA,
    ],
)
def _node_gather(nf_hbm, ne_tab_hbm, nnt_hbm, net_hbm, outn_hbm, oute_hbm,
                 idxn_v, idxe_v, accn_v, acce_v, sem):
    _node_gather_body(nf_hbm, ne_tab_hbm, nnt_hbm, net_hbm, outn_hbm, oute_hbm,
                      idxn_v, idxe_v, accn_v, acce_v, sem)


def _mlp_ln_kernel(nsum, x_ref, w1_ref, b1_ref, w2_ref, b2_ref, g_ref, b_ref,
                   o_ref):
    h1 = jnp.dot(x_ref[...], w1_ref[...],
                 preferred_element_type=jnp.float32) + nsum * b1_ref[...]
    h2 = jax.nn.relu(jnp.dot(h1, w2_ref[...],
                             preferred_element_type=jnp.float32) + b2_ref[...])
    m = jnp.mean(h2, axis=-1, keepdims=True)
    v = jnp.mean((h2 - m) ** 2, axis=-1, keepdims=True)
    o_ref[...] = (h2 - m) / jnp.sqrt(v + 1e-5) * g_ref[...] + b_ref[...]


def _edge_mlp(agg_ef, WeT, be2, WceT, bce2, g2, b2):
    blk = 1000
    grid = E // blk
    return pl.pallas_call(
        functools.partial(_mlp_ln_kernel, float(EDEG + 1)),
        grid=(grid,),
        in_specs=[
            pl.BlockSpec((blk, DE), lambda i: (i, 0)),
            pl.BlockSpec((DE, H), lambda i: (0, 0)),
            pl.BlockSpec((1, H), lambda i: (0, 0)),
            pl.BlockSpec((H, H), lambda i: (0, 0)),
            pl.BlockSpec((1, H), lambda i: (0, 0)),
            pl.BlockSpec((1, H), lambda i: (0, 0)),
            pl.BlockSpec((1, H), lambda i: (0, 0)),
        ],
        out_specs=pl.BlockSpec((blk, H), lambda i: (i, 0)),
        out_shape=jax.ShapeDtypeStruct((E, H), jnp.float32),
    )(agg_ef, WeT, be2, WceT, bce2, g2, b2)


def _node_mlp_kernel(x_ref, e_ref, w1_ref, b1_ref, w2_ref, b2_ref, g_ref,
                     b_ref, o_ref):
    h1 = (jnp.dot(x_ref[...], w1_ref[...], preferred_element_type=jnp.float32)
          + float(DEG + 1) * b1_ref[...] + e_ref[...])
    h2 = jax.nn.relu(jnp.dot(h1, w2_ref[...],
                             preferred_element_type=jnp.float32) + b2_ref[...])
    m = jnp.mean(h2, axis=-1, keepdims=True)
    v = jnp.mean((h2 - m) ** 2, axis=-1, keepdims=True)
    o_ref[...] = (h2 - m) / jnp.sqrt(v + 1e-5) * g_ref[...] + b_ref[...]


def _node_mlp(agg_nf, agg_ne, WnT, bn2, WcnT, bcn2, g2, b2):
    blk = 1000
    grid = N // blk
    return pl.pallas_call(
        _node_mlp_kernel,
        grid=(grid,),
        in_specs=[
            pl.BlockSpec((blk, DN), lambda i: (i, 0)),
            pl.BlockSpec((blk, H), lambda i: (i, 0)),
            pl.BlockSpec((DN, H), lambda i: (0, 0)),
            pl.BlockSpec((1, H), lambda i: (0, 0)),
            pl.BlockSpec((H, H), lambda i: (0, 0)),
            pl.BlockSpec((1, H), lambda i: (0, 0)),
            pl.BlockSpec((1, H), lambda i: (0, 0)),
            pl.BlockSpec((1, H), lambda i: (0, 0)),
        ],
        out_specs=pl.BlockSpec((blk, H), lambda i: (i, 0)),
        out_shape=jax.ShapeDtypeStruct((N, H), jnp.float32),
    )(agg_nf, agg_ne, WnT, bn2, WcnT, bcn2, g2, b2)


def kernel(node_neighbors, edge_neighbors, node_feats, edge_feats,
           We, be, Wn, bn, Wce, bce, Wcn, bcn, ln_g, ln_b):
    en_t = edge_neighbors.astype(jnp.int32).T
    en_t = jnp.pad(en_t, ((0, 0), (0, E_PAD - E)))
    nn_t = node_neighbors[:, :, 0].astype(jnp.int32).T
    ne_t = node_neighbors[:, :, 1].astype(jnp.int32).T
    nn_t = jnp.pad(nn_t, ((0, 0), (0, N_PAD - N)))
    ne_t = jnp.pad(ne_t, ((0, 0), (0, N_PAD - N)))
    ef_pad = jnp.pad(edge_feats, ((0, E_PAD - E), (0, 0)))
    nf_pad = jnp.pad(node_feats, ((0, N_PAD - N), (0, 0)))

    be2 = be.reshape(1, H)
    bn2 = bn.reshape(1, H)
    bce2 = bce.reshape(1, H)
    bcn2 = bcn.reshape(1, H)
    g2 = ln_g.reshape(1, H)
    b2 = ln_b.reshape(1, H)

    agg_ef = _edge_gather(ef_pad, en_t)[:E]

    new_edge = _edge_mlp(agg_ef, We.T, be2, Wce.T, bce2, g2, b2)

    agg_nf, agg_ne = _node_gather(nf_pad, new_edge, nn_t, ne_t)

    new_node = _node_mlp(agg_nf[:N], agg_ne[:N], Wn.T, bn2, Wcn.T, bcn2,
                         g2, b2)

    return (new_node, new_edge)
